# Initial kernel scaffold; baseline (speedup 1.0000x reference)
import jax
import jax.numpy as jnp
from jax.experimental import pallas as pl


def kernel(agenttypes, ctrs, xt_enc, edge_index, W_at, g_at, b_at, W_d, g_d, b_d, W_f, bias_f, g_f, bln_f, W1, g1, b1, W2, g2, b2, W_out, b_out):
    raise NotImplementedError("write your pallas kernel here")


# speedup vs baseline: 5.6936x; 5.6936x over previous
import functools

import jax
import jax.numpy as jnp
import numpy as np
from jax.experimental import pallas as pl
from jax.experimental.pallas import tpu as pltpu
from jax.experimental.pallas import tpu_sc as plsc

N = 10000
E = 160000
H = 128
NUM_AT = 2
NUM_ET = 3
NG = 32

D_PACK = 144
GATHER_W = 128
TC_BLOCK = 2000
EPS = 1e-5


def _sc_gather(table, src, dst):
    mesh = plsc.VectorSubcoreMesh(core_axis_name="c", subcore_axis_name="s")
    out_t = jax.ShapeDtypeStruct((E, D_PACK), jnp.float32)

    @functools.partial(
        pl.kernel, out_type=[out_t, out_t], mesh=mesh,
        compiler_params=pltpu.CompilerParams(use_tc_tiling_on_sc=False))
    def gather_kernel(table_hbm, src_hbm, dst_hbm, os_hbm, od_hbm):
        def body(src_v, dst_v, os_v, od_v):
            pltpu.sync_copy(table_hbm.at[src_v.at[0]], os_v)
            pltpu.sync_copy(table_hbm.at[dst_v.at[0]], od_v)

        pltpu.emit_pipeline(
            body,
            grid=(E // GATHER_W,),
            in_specs=[
                pl.BlockSpec((1, GATHER_W), lambda i: (0, i)),
                pl.BlockSpec((1, GATHER_W), lambda i: (0, i)),
            ],
            out_specs=[
                pl.BlockSpec((GATHER_W, D_PACK), lambda i: (i, 0)),
                pl.BlockSpec((GATHER_W, D_PACK), lambda i: (i, 0)),
            ],
            core_axis_name=("c", "s"),
            dimension_semantics=(pltpu.PARALLEL,),
        )(src_hbm, dst_hbm, os_hbm, od_hbm)

    return gather_kernel(table, src, dst)


def _tc_body(xs_ref, xd_ref, wsm_ref, wft_ref, w1t_ref, w2t_ref, wot_ref,
             mg_ref, p_ref, bout_ref, out_ref):
    f32 = jnp.float32
    xs = xs_ref[...]
    xd = xd_ref[...]
    mg = mg_ref[...]
    p = p_ref[...]

    def gn(x, g, b):
        m = jnp.dot(x, mg, preferred_element_type=f32)
        xc = x - m
        v = jnp.dot(xc * xc, mg, preferred_element_type=f32)
        return xc * jax.lax.rsqrt(v + EPS) * g + b

    sml = jnp.concatenate([xs[:, H:H + 16], xd[:, H:H + 16]], axis=1)
    lin = jnp.dot(sml, wsm_ref[...], preferred_element_type=f32)
    at_enc = jnp.maximum(gn(lin[:, :H], p[0:1], p[1:2]), 0.0)
    dist = jnp.maximum(gn(lin[:, H:], p[2:3], p[3:4]), 0.0)

    wft = wft_ref[...]
    h = (jnp.dot(xs[:, :H], wft[0:H], preferred_element_type=f32)
         + jnp.dot(xd[:, :H], wft[H:2 * H], preferred_element_type=f32)
         + jnp.dot(dist, wft[2 * H:3 * H], preferred_element_type=f32)
         + jnp.dot(at_enc, wft[3 * H:], preferred_element_type=f32)
         + p[4:5])
    m = jnp.mean(h, axis=1, keepdims=True)
    hc = h - m
    v = jnp.mean(hc * hc, axis=1, keepdims=True)
    h1 = jnp.maximum(hc * jax.lax.rsqrt(v + EPS) * p[5:6] + p[6:7], 0.0)

    t = jnp.maximum(gn(jnp.dot(h1, w1t_ref[...], preferred_element_type=f32),
                       p[7:8], p[8:9]), 0.0)
    t = gn(jnp.dot(t, w2t_ref[...], preferred_element_type=f32),
           p[9:10], p[10:11])
    t = jnp.maximum(t + h1, 0.0)
    out_ref[...] = (jnp.dot(t, wot_ref[...], preferred_element_type=f32)
                    + bout_ref[...])


def kernel(agenttypes, ctrs, xt_enc, edge_index, W_at, g_at, b_at, W_d, g_d,
           b_d, W_f, bias_f, g_f, bln_f, W1, g1, b1, W2, g2, b2, W_out,
           b_out):
    table = jnp.concatenate(
        [xt_enc, agenttypes, ctrs,
         jnp.zeros((N, D_PACK - H - 4), jnp.float32)], axis=1)
    src = edge_index[0].reshape(1, E)
    dst = edge_index[1].reshape(1, E)

    wsm = jnp.zeros((32, 2 * H), jnp.float32)
    wsm = wsm.at[0:2, :H].set(W_at[:, 0:NUM_AT].T)
    wsm = wsm.at[16:18, :H].set(W_at[:, NUM_AT:].T)
    wsm = wsm.at[2:4, H:].set(-W_d.T)
    wsm = wsm.at[18:20, H:].set(W_d.T)

    wft = W_f.T
    w1t = W1.T
    w2t = W2.T
    wot = jnp.zeros((H, 8), jnp.float32).at[:, :NUM_ET].set(W_out.T)
    bout = jnp.zeros((1, 8), jnp.float32).at[0, :NUM_ET].set(b_out)

    mg = jnp.asarray(np.kron(np.eye(NG, dtype=np.float32),
                             np.full((H // NG, H // NG), NG / H,
                                     dtype=np.float32)))

    p = jnp.zeros((16, H), jnp.float32)
    for i, vec in enumerate([g_at, b_at, g_d, b_d, bias_f, g_f, bln_f,
                             g1, b1, g2, b2]):
        p = p.at[i].set(vec)

    xs, xd = _sc_gather(table, src, dst)

    nb = E // TC_BLOCK
    full = lambda shape: pl.BlockSpec(shape, lambda i: (0, 0))
    out8 = pl.pallas_call(
        _tc_body,
        grid=(nb,),
        in_specs=[
            pl.BlockSpec((TC_BLOCK, D_PACK), lambda i: (i, 0)),
            pl.BlockSpec((TC_BLOCK, D_PACK), lambda i: (i, 0)),
            full((32, 2 * H)),
            full((4 * H, H)),
            full((H, H)),
            full((H, H)),
            full((H, 8)),
            full((H, H)),
            full((16, H)),
            full((1, 8)),
        ],
        out_specs=pl.BlockSpec((TC_BLOCK, 8), lambda i: (i, 0)),
        out_shape=jax.ShapeDtypeStruct((E, 8), jnp.float32),
        compiler_params=pltpu.CompilerParams(
            dimension_semantics=("parallel",)),
    )(xs, xd, wsm, wft, w1t, w2t, wot, mg, p, bout)

    return out8[:, :NUM_ET]


# speedup vs baseline: 6.8355x; 1.2005x over previous
import functools

import jax
import jax.numpy as jnp
import numpy as np
from jax.experimental import pallas as pl
from jax.experimental.pallas import tpu as pltpu
from jax.experimental.pallas import tpu_sc as plsc

N = 10000
E = 160000
H = 128
NUM_AT = 2
NUM_ET = 3
NG = 32

D_PACK = 144
GATHER_W = 128
TC_BLOCK = 2000
EPS = 1e-5


def _sc_gather(table, small, src, dst):
    mesh = plsc.VectorSubcoreMesh(core_axis_name="c", subcore_axis_name="s")
    out_big = jax.ShapeDtypeStruct((E, H), jnp.float32)
    out_sml = jax.ShapeDtypeStruct((E, 16), jnp.float32)

    @functools.partial(
        pl.kernel, out_type=[out_big, out_big, out_sml, out_sml], mesh=mesh,
        compiler_params=pltpu.CompilerParams(use_tc_tiling_on_sc=False))
    def gather_kernel(table_hbm, small_hbm, src_hbm, dst_hbm,
                      os_hbm, od_hbm, ss_hbm, sd_hbm):
        def body(src_v, dst_v, os_v, od_v, ss_v, sd_v):
            pltpu.sync_copy(table_hbm.at[src_v.at[0]], os_v)
            pltpu.sync_copy(table_hbm.at[dst_v.at[0]], od_v)
            pltpu.sync_copy(small_hbm.at[src_v.at[0]], ss_v)
            pltpu.sync_copy(small_hbm.at[dst_v.at[0]], sd_v)

        pltpu.emit_pipeline(
            body,
            grid=(E // GATHER_W,),
            in_specs=[
                pl.BlockSpec((1, GATHER_W), lambda i: (0, i)),
                pl.BlockSpec((1, GATHER_W), lambda i: (0, i)),
            ],
            out_specs=[
                pl.BlockSpec((GATHER_W, H), lambda i: (i, 0)),
                pl.BlockSpec((GATHER_W, H), lambda i: (i, 0)),
                pl.BlockSpec((GATHER_W, 16), lambda i: (i, 0)),
                pl.BlockSpec((GATHER_W, 16), lambda i: (i, 0)),
            ],
            core_axis_name=("c", "s"),
            dimension_semantics=(pltpu.PARALLEL,),
        )(src_hbm, dst_hbm, os_hbm, od_hbm, ss_hbm, sd_hbm)

    return gather_kernel(table, small, src, dst)


def _tc_body(xs_ref, xd_ref, ss_ref, sd_ref, wsm_ref, wft_ref, w1t_ref,
             w2t_ref, wot_ref, mg_ref, p_ref, bout_ref, out_ref):
    f32 = jnp.float32
    xs = xs_ref[...]
    xd = xd_ref[...]
    mg = mg_ref[...]
    p = p_ref[...]

    def gn(x, g, b):
        m = jnp.dot(x, mg, preferred_element_type=f32)
        xc = x - m
        v = jnp.dot(xc * xc, mg, preferred_element_type=f32)
        return xc * jax.lax.rsqrt(v + EPS) * g + b

    sml = jnp.concatenate([ss_ref[...], sd_ref[...]], axis=1)
    lin = jnp.dot(sml, wsm_ref[...], preferred_element_type=f32)
    at_enc = jnp.maximum(gn(lin[:, :H], p[0:1], p[1:2]), 0.0)
    dist = jnp.maximum(gn(lin[:, H:], p[2:3], p[3:4]), 0.0)

    wft = wft_ref[...]
    h = (jnp.dot(xs, wft[0:H], preferred_element_type=f32)
         + jnp.dot(xd, wft[H:2 * H], preferred_element_type=f32)
         + jnp.dot(dist, wft[2 * H:3 * H], preferred_element_type=f32)
         + jnp.dot(at_enc, wft[3 * H:], preferred_element_type=f32)
         + p[4:5])
    m = jnp.mean(h, axis=1, keepdims=True)
    hc = h - m
    v = jnp.mean(hc * hc, axis=1, keepdims=True)
    h1 = jnp.maximum(hc * jax.lax.rsqrt(v + EPS) * p[5:6] + p[6:7], 0.0)

    t = jnp.maximum(gn(jnp.dot(h1, w1t_ref[...], preferred_element_type=f32),
                       p[7:8], p[8:9]), 0.0)
    t = gn(jnp.dot(t, w2t_ref[...], preferred_element_type=f32),
           p[9:10], p[10:11])
    t = jnp.maximum(t + h1, 0.0)
    out_ref[...] = (jnp.dot(t, wot_ref[...], preferred_element_type=f32)
                    + bout_ref[...])


def kernel(agenttypes, ctrs, xt_enc, edge_index, W_at, g_at, b_at, W_d, g_d,
           b_d, W_f, bias_f, g_f, bln_f, W1, g1, b1, W2, g2, b2, W_out,
           b_out):
    small = jnp.concatenate(
        [agenttypes, ctrs, jnp.zeros((N, 12), jnp.float32)], axis=1)
    src = edge_index[0].reshape(1, E)
    dst = edge_index[1].reshape(1, E)

    wsm = jnp.zeros((32, 2 * H), jnp.float32)
    wsm = wsm.at[0:2, :H].set(W_at[:, 0:NUM_AT].T)
    wsm = wsm.at[16:18, :H].set(W_at[:, NUM_AT:].T)
    wsm = wsm.at[2:4, H:].set(-W_d.T)
    wsm = wsm.at[18:20, H:].set(W_d.T)

    wft = W_f.T
    w1t = W1.T
    w2t = W2.T
    wot = jnp.zeros((H, 8), jnp.float32).at[:, :NUM_ET].set(W_out.T)
    bout = jnp.zeros((1, 8), jnp.float32).at[0, :NUM_ET].set(b_out)

    mg = jnp.asarray(np.kron(np.eye(NG, dtype=np.float32),
                             np.full((H // NG, H // NG), NG / H,
                                     dtype=np.float32)))

    p = jnp.zeros((16, H), jnp.float32)
    for i, vec in enumerate([g_at, b_at, g_d, b_d, bias_f, g_f, bln_f,
                             g1, b1, g2, b2]):
        p = p.at[i].set(vec)

    xs, xd, ss, sd = _sc_gather(xt_enc, small, src, dst)

    nb = E // TC_BLOCK
    full = lambda shape: pl.BlockSpec(shape, lambda i: (0, 0))
    out8 = pl.pallas_call(
        _tc_body,
        grid=(nb,),
        in_specs=[
            pl.BlockSpec((TC_BLOCK, H), lambda i: (i, 0)),
            pl.BlockSpec((TC_BLOCK, H), lambda i: (i, 0)),
            pl.BlockSpec((TC_BLOCK, 16), lambda i: (i, 0)),
            pl.BlockSpec((TC_BLOCK, 16), lambda i: (i, 0)),
            full((32, 2 * H)),
            full((4 * H, H)),
            full((H, H)),
            full((H, H)),
            full((H, 8)),
            full((H, H)),
            full((16, H)),
            full((1, 8)),
        ],
        out_specs=pl.BlockSpec((TC_BLOCK, 8), lambda i: (i, 0)),
        out_shape=jax.ShapeDtypeStruct((E, 8), jnp.float32),
        compiler_params=pltpu.CompilerParams(
            dimension_semantics=("parallel",)),
    )(xs, xd, ss, sd, wsm, wft, w1t, w2t, wot, mg, p, bout)

    return out8[:, :NUM_ET]


# speedup vs baseline: 7.3200x; 1.0709x over previous
import functools

import jax
import jax.numpy as jnp
import numpy as np
from jax.experimental import pallas as pl
from jax.experimental.pallas import tpu as pltpu
from jax.experimental.pallas import tpu_sc as plsc

N = 10000
E = 160000
H = 128
NUM_AT = 2
NUM_ET = 3
NG = 32

GATHER_W = 128
TC_BLOCK = 2000
PREP_BLOCK = 2000
EPS = 1e-5


def _sc_gather(table_u, table_v, small, src, dst):
    mesh = plsc.VectorSubcoreMesh(core_axis_name="c", subcore_axis_name="s")
    out_big = jax.ShapeDtypeStruct((E, H), jnp.float32)
    out_sml = jax.ShapeDtypeStruct((E, 16), jnp.float32)

    @functools.partial(
        pl.kernel, out_type=[out_big, out_big, out_sml, out_sml], mesh=mesh,
        compiler_params=pltpu.CompilerParams(use_tc_tiling_on_sc=False))
    def gather_kernel(u_hbm, v_hbm, small_hbm, src_hbm, dst_hbm,
                      os_hbm, od_hbm, ss_hbm, sd_hbm):
        def body(src_v, dst_v, os_v, od_v, ss_v, sd_v):
            pltpu.sync_copy(u_hbm.at[src_v.at[0]], os_v)
            pltpu.sync_copy(v_hbm.at[dst_v.at[0]], od_v)
            pltpu.sync_copy(small_hbm.at[src_v.at[0]], ss_v)
            pltpu.sync_copy(small_hbm.at[dst_v.at[0]], sd_v)

        pltpu.emit_pipeline(
            body,
            grid=(E // GATHER_W,),
            in_specs=[
                pl.BlockSpec((1, GATHER_W), lambda i: (0, i)),
                pl.BlockSpec((1, GATHER_W), lambda i: (0, i)),
            ],
            out_specs=[
                pl.BlockSpec((GATHER_W, H), lambda i: (i, 0)),
                pl.BlockSpec((GATHER_W, H), lambda i: (i, 0)),
                pl.BlockSpec((GATHER_W, 16), lambda i: (i, 0)),
                pl.BlockSpec((GATHER_W, 16), lambda i: (i, 0)),
            ],
            core_axis_name=("c", "s"),
            dimension_semantics=(pltpu.PARALLEL,),
        )(src_hbm, dst_hbm, os_hbm, od_hbm, ss_hbm, sd_hbm)

    return gather_kernel(table_u, table_v, small, src, dst)


def _prep_body(xt_ref, wf12_ref, u_ref, v_ref):
    uc = jnp.dot(xt_ref[...], wf12_ref[...],
                 preferred_element_type=jnp.float32)
    u_ref[...] = uc[:, :H]
    v_ref[...] = uc[:, H:]


def _tc_body(xs_ref, xd_ref, ss_ref, sd_ref, wsm_ref, mg2_ref, mg_ref,
             wcad_ref, w1c_ref, w2c_ref, wot_ref, p_ref, p2_ref, bout_ref,
             out_ref):
    f32 = jnp.float32
    mg = mg_ref[...]
    p = p_ref[...]
    p2 = p2_ref[...]

    sml = jnp.concatenate([ss_ref[...], sd_ref[...]], axis=1)
    lin = jnp.dot(sml, wsm_ref[...], preferred_element_type=f32)
    m = jnp.dot(lin, mg2_ref[...], preferred_element_type=f32)
    xc = lin - m
    v = jnp.dot(xc * xc, mg2_ref[...], preferred_element_type=f32)
    ad = jnp.maximum(xc * jax.lax.rsqrt(v + EPS) * p2[0:1] + p2[1:2], 0.0)

    h = (xs_ref[...] + xd_ref[...]
         + jnp.dot(ad, wcad_ref[...], preferred_element_type=f32)
         + p[4:5])
    mh = jnp.mean(h, axis=1, keepdims=True)
    hc = h - mh
    vh = jnp.mean(hc * hc, axis=1, keepdims=True)
    h1 = jnp.maximum(hc * jax.lax.rsqrt(vh + EPS) * p[5:6] + p[6:7], 0.0)

    t1c = jnp.dot(h1, w1c_ref[...], preferred_element_type=f32)
    xc1 = t1c[:, :H] - t1c[:, H:]
    v1 = jnp.dot(xc1 * xc1, mg, preferred_element_type=f32)
    t1 = jnp.maximum(xc1 * jax.lax.rsqrt(v1 + EPS) * p[7:8] + p[8:9], 0.0)

    t2c = jnp.dot(t1, w2c_ref[...], preferred_element_type=f32)
    xc2 = t2c[:, :H] - t2c[:, H:]
    v2 = jnp.dot(xc2 * xc2, mg, preferred_element_type=f32)
    t2 = xc2 * jax.lax.rsqrt(v2 + EPS) * p[9:10] + p[10:11]

    t = jnp.maximum(t2 + h1, 0.0)
    out_ref[...] = (jnp.dot(t, wot_ref[...], preferred_element_type=f32)
                    + bout_ref[...])


def kernel(agenttypes, ctrs, xt_enc, edge_index, W_at, g_at, b_at, W_d, g_d,
           b_d, W_f, bias_f, g_f, bln_f, W1, g1, b1, W2, g2, b2, W_out,
           b_out):
    small = jnp.concatenate(
        [agenttypes, ctrs, jnp.zeros((N, 12), jnp.float32)], axis=1)
    src = edge_index[0].reshape(1, E)
    dst = edge_index[1].reshape(1, E)

    wsm = jnp.zeros((32, 2 * H), jnp.float32)
    wsm = wsm.at[0:2, :H].set(W_at[:, 0:NUM_AT].T)
    wsm = wsm.at[16:18, :H].set(W_at[:, NUM_AT:].T)
    wsm = wsm.at[2:4, H:].set(-W_d.T)
    wsm = wsm.at[18:20, H:].set(W_d.T)

    mg_np = np.kron(np.eye(NG, dtype=np.float32),
                    np.full((H // NG, H // NG), NG / H, dtype=np.float32))
    mg = jnp.asarray(mg_np)
    mg2 = jnp.asarray(np.kron(np.eye(2, dtype=np.float32), mg_np))

    wf12 = jnp.concatenate([W_f[:, 0:H].T, W_f[:, H:2 * H].T], axis=1)
    wcad = jnp.concatenate([W_f[:, 3 * H:].T, W_f[:, 2 * H:3 * H].T], axis=0)
    w1c = jnp.concatenate([W1.T, W1.T @ mg], axis=1)
    w2c = jnp.concatenate([W2.T, W2.T @ mg], axis=1)
    wot = jnp.zeros((H, 8), jnp.float32).at[:, :NUM_ET].set(W_out.T)
    bout = jnp.zeros((1, 8), jnp.float32).at[0, :NUM_ET].set(b_out)

    p = jnp.zeros((16, H), jnp.float32)
    for i, vec in enumerate([g_at, b_at, g_d, b_d, bias_f, g_f, bln_f,
                             g1, b1, g2, b2]):
        p = p.at[i].set(vec)
    p2 = jnp.stack([jnp.concatenate([g_at, g_d]),
                    jnp.concatenate([b_at, b_d])])

    u, v = pl.pallas_call(
        _prep_body,
        grid=(N // PREP_BLOCK,),
        in_specs=[
            pl.BlockSpec((PREP_BLOCK, H), lambda i: (i, 0)),
            pl.BlockSpec((H, 2 * H), lambda i: (0, 0)),
        ],
        out_specs=[
            pl.BlockSpec((PREP_BLOCK, H), lambda i: (i, 0)),
            pl.BlockSpec((PREP_BLOCK, H), lambda i: (i, 0)),
        ],
        out_shape=[jax.ShapeDtypeStruct((N, H), jnp.float32),
                   jax.ShapeDtypeStruct((N, H), jnp.float32)],
        compiler_params=pltpu.CompilerParams(
            dimension_semantics=("parallel",)),
    )(xt_enc, wf12)

    xs, xd, ss, sd = _sc_gather(u, v, small, src, dst)

    nb = E // TC_BLOCK
    full = lambda shape: pl.BlockSpec(shape, lambda i: (0, 0))
    out8 = pl.pallas_call(
        _tc_body,
        grid=(nb,),
        in_specs=[
            pl.BlockSpec((TC_BLOCK, H), lambda i: (i, 0)),
            pl.BlockSpec((TC_BLOCK, H), lambda i: (i, 0)),
            pl.BlockSpec((TC_BLOCK, 16), lambda i: (i, 0)),
            pl.BlockSpec((TC_BLOCK, 16), lambda i: (i, 0)),
            full((32, 2 * H)),
            full((2 * H, 2 * H)),
            full((H, H)),
            full((2 * H, H)),
            full((H, 2 * H)),
            full((H, 2 * H)),
            full((H, 8)),
            full((16, H)),
            full((2, 2 * H)),
            full((1, 8)),
        ],
        out_specs=pl.BlockSpec((TC_BLOCK, 8), lambda i: (i, 0)),
        out_shape=jax.ShapeDtypeStruct((E, 8), jnp.float32),
        compiler_params=pltpu.CompilerParams(
            dimension_semantics=("parallel",)),
    )(xs, xd, ss, sd, wsm, mg2, mg, wcad, w1c, w2c, wot, p, p2, bout)

    return out8[:, :NUM_ET]


# speedup vs baseline: 9.5653x; 1.3067x over previous
import functools

import jax
import jax.numpy as jnp
import numpy as np
from jax.experimental import pallas as pl
from jax.experimental.pallas import tpu as pltpu
from jax.experimental.pallas import tpu_sc as plsc

N = 10000
E = 160000
H = 128
D = 256
NUM_AT = 2
NUM_ET = 3
NG = 32

CHUNKS = 5
EC = E // CHUNKS
GATHER_W = 128
TC_BLOCK = 3200
PREP_BLOCK = 2000
EPS = 1e-5


def _sc_gather(w_tab, idx):
    mesh = plsc.VectorSubcoreMesh(core_axis_name="c", subcore_axis_name="s")
    out_t = jax.ShapeDtypeStruct((2 * EC, D), jnp.float32)

    @functools.partial(pl.kernel, out_type=out_t, mesh=mesh)
    def gather_kernel(w_hbm, idx_hbm, o_hbm):
        def body(idx_v, o_v):
            pltpu.sync_copy(w_hbm.at[idx_v.at[0]], o_v)

        pltpu.emit_pipeline(
            body,
            grid=(2 * EC // GATHER_W,),
            in_specs=[pl.BlockSpec((1, GATHER_W), lambda i: (0, i))],
            out_specs=[pl.BlockSpec((GATHER_W, D), lambda i: (i, 0))],
            core_axis_name=("c", "s"),
            dimension_semantics=(pltpu.PARALLEL,),
        )(idx_hbm, o_hbm)

    return gather_kernel(w_tab, idx)


def _prep_body(xt_ref, small_ref, wf12_ref, w_ref):
    i = pl.program_id(0)
    uc = jnp.dot(xt_ref[...], wf12_ref[...],
                 preferred_element_type=jnp.float32)
    proj = jnp.where(i >= N // PREP_BLOCK, uc[:, H:], uc[:, :H])
    pad = jnp.zeros((xt_ref.shape[0], D - H - 16), jnp.float32)
    w_ref[...] = jnp.concatenate([proj, small_ref[...], pad], axis=1)


def _tc_body(xs_ref, xd_ref, wsm_ref, mg2_ref, wcad_ref, w1c_ref, w2c_ref,
             wot_ref, p_ref, p2_ref, bout_ref, out_ref):
    f32 = jnp.float32
    p = p_ref[...]
    p2 = p2_ref[...]
    xs = xs_ref[...]
    xd = xd_ref[...]

    sml = jnp.concatenate([xs[:, H:H + 16], xd[:, H:H + 16]], axis=1)
    xc = jnp.dot(sml, wsm_ref[...], preferred_element_type=f32)
    mad = jnp.dot(xc, mg2_ref[...], preferred_element_type=f32)
    vad = jnp.dot(xc * xc, mg2_ref[...], preferred_element_type=f32)
    ad = jnp.maximum(
        (xc - mad) * jax.lax.rsqrt(vad - mad * mad + EPS) * p2[0:1]
        + p2[1:2], 0.0)

    h = (xs[:, :H] + xd[:, :H]
         + jnp.dot(ad, wcad_ref[...], preferred_element_type=f32)
         + p[4:5])
    mh = jnp.mean(h, axis=1, keepdims=True)
    hc = h - mh
    vh = jnp.mean(hc * hc, axis=1, keepdims=True)
    h1 = jnp.maximum(hc * jax.lax.rsqrt(vh + EPS) * p[5:6] + p[6:7], 0.0)

    def gn_stats(x):
        s = jnp.dot(jnp.concatenate([x * x, x], axis=1), mg2_ref[...],
                    preferred_element_type=f32)
        v, m = s[:, :H], s[:, H:]
        return (x - m) * jax.lax.rsqrt(v - m * m + EPS)

    xc1 = jnp.dot(h1, w1c_ref[...], preferred_element_type=f32)
    t1 = jnp.maximum(gn_stats(xc1) * p[7:8] + p[8:9], 0.0)

    xc2 = jnp.dot(t1, w2c_ref[...], preferred_element_type=f32)
    t2 = gn_stats(xc2) * p[9:10] + p[10:11]

    t = jnp.maximum(t2 + h1, 0.0)
    out_ref[...] = (jnp.dot(t, wot_ref[...], preferred_element_type=f32)
                    + bout_ref[...])


def kernel(agenttypes, ctrs, xt_enc, edge_index, W_at, g_at, b_at, W_d, g_d,
           b_d, W_f, bias_f, g_f, bln_f, W1, g1, b1, W2, g2, b2, W_out,
           b_out):
    small = jnp.concatenate(
        [agenttypes, ctrs, jnp.zeros((N, 12), jnp.float32)], axis=1)
    src = edge_index[0].reshape(1, E)
    dst = edge_index[1].reshape(1, E)

    wsm = jnp.zeros((32, 2 * H), jnp.float32)
    wsm = wsm.at[0:2, :H].set(W_at[:, 0:NUM_AT].T)
    wsm = wsm.at[16:18, :H].set(W_at[:, NUM_AT:].T)
    wsm = wsm.at[2:4, H:].set(-W_d.T)
    wsm = wsm.at[18:20, H:].set(W_d.T)

    mg_np = np.kron(np.eye(NG, dtype=np.float32),
                    np.full((H // NG, H // NG), NG / H, dtype=np.float32))
    mg = jnp.asarray(mg_np)
    mg2 = jnp.asarray(np.kron(np.eye(2, dtype=np.float32), mg_np))

    wsm = wsm - wsm @ mg2

    wf12 = jnp.concatenate([W_f[:, 0:H].T, W_f[:, H:2 * H].T], axis=1)
    wcad = jnp.concatenate([W_f[:, 3 * H:].T, W_f[:, 2 * H:3 * H].T], axis=0)
    w1c = W1.T - W1.T @ mg
    w2c = W2.T - W2.T @ mg
    wot = jnp.zeros((H, 8), jnp.float32).at[:, :NUM_ET].set(W_out.T)
    bout = jnp.zeros((1, 8), jnp.float32).at[0, :NUM_ET].set(b_out)

    p = jnp.zeros((16, H), jnp.float32)
    for i, vec in enumerate([g_at, b_at, g_d, b_d, bias_f, g_f, bln_f,
                             g1, b1, g2, b2]):
        p = p.at[i].set(vec)
    p2 = jnp.stack([jnp.concatenate([g_at, g_d]),
                    jnp.concatenate([b_at, b_d])])

    npb = N // PREP_BLOCK
    w_tab = pl.pallas_call(
        _prep_body,
        grid=(2 * npb,),
        in_specs=[
            pl.BlockSpec((PREP_BLOCK, H), lambda i: (i % npb, 0)),
            pl.BlockSpec((PREP_BLOCK, 16), lambda i: (i % npb, 0)),
            pl.BlockSpec((H, 2 * H), lambda i: (0, 0)),
        ],
        out_specs=pl.BlockSpec((PREP_BLOCK, D), lambda i: (i, 0)),
        out_shape=jax.ShapeDtypeStruct((2 * N, D), jnp.float32),
        compiler_params=pltpu.CompilerParams(
            dimension_semantics=("arbitrary",)),
    )(xt_enc, small, wf12)

    full = lambda shape: pl.BlockSpec(shape, lambda i: (0, 0))
    tc_call = pl.pallas_call(
        _tc_body,
        grid=(EC // TC_BLOCK,),
        in_specs=[
            pl.BlockSpec((TC_BLOCK, D), lambda i: (i, 0)),
            pl.BlockSpec((TC_BLOCK, D), lambda i: (EC // TC_BLOCK + i, 0)),
            full((32, 2 * H)),
            full((2 * H, 2 * H)),
            full((2 * H, H)),
            full((H, H)),
            full((H, H)),
            full((H, 8)),
            full((16, H)),
            full((2, 2 * H)),
            full((1, 8)),
        ],
        out_specs=pl.BlockSpec((TC_BLOCK, 8), lambda i: (i, 0)),
        out_shape=jax.ShapeDtypeStruct((EC, 8), jnp.float32),
        compiler_params=pltpu.CompilerParams(
            dimension_semantics=("parallel",)),
    )

    dstp = dst + N
    outs = []
    for k in range(CHUNKS):
        sk = jax.lax.slice(src, (0, k * EC), (1, (k + 1) * EC))
        dk = jax.lax.slice(dstp, (0, k * EC), (1, (k + 1) * EC))
        gathered = _sc_gather(w_tab, jnp.concatenate([sk, dk], axis=1))
        outs.append(tc_call(gathered, gathered, wsm, mg2, wcad, w1c, w2c,
                            wot, p, p2, bout))

    return jnp.concatenate(outs, axis=0)[:, :NUM_ET]
